# baseline (speedup 1.0000x reference)
import functools

import jax
import jax.numpy as jnp
from jax import lax
from jax.experimental import pallas as pl
from jax.experimental.pallas import tpu as pltpu
from jax.experimental.pallas import tpu_sc as plsc

N = 10000
E = 160000
D = 256
H = 4
C = 256
HC = H * C
G = 16
FC1 = 512
FC2 = 128

NB = 1000
NBLK = N // NB
KB = 8
NW = 32
EVALID = E // NW
EWP = 5120
GRP = 313
AW = 16
FBG = 32


def _tc0_body(x_ref, W_ref, fs_ref, fd_ref, xl_ref, atab_ref):
    xb = x_ref[...]
    xl = jnp.dot(xb, W_ref[...], preferred_element_type=jnp.float32)
    xl_ref[...] = xl.reshape(NB, KB, 128).transpose(1, 0, 2)
    rows = lax.broadcasted_iota(jnp.int32, (HC, AW), 0) // C
    cols = lax.broadcasted_iota(jnp.int32, (HC, AW), 1)
    fs = fs_ref[...].reshape(HC, 1)
    fd = fd_ref[...].reshape(HC, 1)
    WS = jnp.where(cols == rows, fs, 0.0) + jnp.where(cols == rows + H, fd, 0.0)
    atab_ref[...] = jnp.dot(xl, WS, preferred_element_type=jnp.float32)


def _tc0(x, W0, fs0, fd0):
    return pl.pallas_call(
        _tc0_body,
        grid=(NBLK,),
        in_specs=[
            pl.BlockSpec((NB, D), lambda i: (i, 0)),
            pl.BlockSpec((D, HC), lambda i: (0, 0)),
            pl.BlockSpec((1, HC), lambda i: (0, 0)),
            pl.BlockSpec((1, HC), lambda i: (0, 0)),
        ],
        out_specs=[
            pl.BlockSpec((KB, NB, 128), lambda i: (0, i, 0)),
            pl.BlockSpec((NB, AW), lambda i: (i, 0)),
        ],
        out_shape=[
            jax.ShapeDtypeStruct((KB, N, 128), jnp.float32),
            jax.ShapeDtypeStruct((N, AW), jnp.float32),
        ],
    )(x, W0, fs0.reshape(1, HC), fd0.reshape(1, HC))


def _tc1_body(p_ref, den_ref, b_ref, W_ref, fs_ref, fd_ref,
              xl_ref, atab_ref, acc_ref):
    j = pl.program_id(1)

    @pl.when(j == 0)
    def _():
        acc_ref[...] = jnp.zeros_like(acc_ref)

    p = p_ref[...]
    psum = p[0, 0] + p[1, 0]
    den = den_ref[...]
    d4 = den[0] + den[1]
    h_idx = j // 2
    cols = lax.broadcasted_iota(jnp.int32, (NB, H), 1)
    dh = jnp.sum(jnp.where(cols == h_idx, d4, 0.0), axis=1, keepdims=True)
    normed = psum * (1.0 / (dh + 1e-30)) + b_ref[...]
    hblk = jnp.where(normed > 0, normed, jnp.exp(normed) - 1.0)
    acc_ref[...] += jnp.dot(hblk, W_ref[...], preferred_element_type=jnp.float32)

    @pl.when(j == KB - 1)
    def _():
        xl1 = acc_ref[...]
        xl_ref[...] = xl1.reshape(NB, KB, 128).transpose(1, 0, 2)
        rows = lax.broadcasted_iota(jnp.int32, (HC, AW), 0) // C
        wcols = lax.broadcasted_iota(jnp.int32, (HC, AW), 1)
        fs = fs_ref[...].reshape(HC, 1)
        fd = fd_ref[...].reshape(HC, 1)
        WS = jnp.where(wcols == rows, fs, 0.0) \
           + jnp.where(wcols == rows + H, fd, 0.0)
        atab_ref[...] = jnp.dot(xl1, WS, preferred_element_type=jnp.float32)


def _tc1(partial, denom, b0, W1, fs1, fd1):
    return pl.pallas_call(
        _tc1_body,
        grid=(NBLK, KB),
        in_specs=[
            pl.BlockSpec((2, 1, NB, 128), lambda i, j: (0, j, i, 0)),
            pl.BlockSpec((2, NB, H), lambda i, j: (0, i, 0)),
            pl.BlockSpec((1, 128), lambda i, j: (0, j)),
            pl.BlockSpec((128, HC), lambda i, j: (j, 0)),
            pl.BlockSpec((1, HC), lambda i, j: (0, 0)),
            pl.BlockSpec((1, HC), lambda i, j: (0, 0)),
        ],
        out_specs=[
            pl.BlockSpec((KB, NB, 128), lambda i, j: (0, i, 0)),
            pl.BlockSpec((NB, AW), lambda i, j: (i, 0)),
        ],
        out_shape=[
            jax.ShapeDtypeStruct((KB, N, 128), jnp.float32),
            jax.ShapeDtypeStruct((N, AW), jnp.float32),
        ],
        scratch_shapes=[
            pltpu.VMEM((NB, HC), jnp.float32),
        ],
    )(partial, denom, b0.reshape(1, HC), W1,
      fs1.reshape(1, HC), fd1.reshape(1, HC))


def _tc2_body(p_ref, den_ref, b_ref, batch_ref, fcW0_ref, fcb0_ref,
              fcW1_ref, fcb1_ref, emb_ref, out_ref, pool_ref):
    i = pl.program_id(0)

    @pl.when(i == 0)
    def _():
        pool_ref[...] = jnp.zeros_like(pool_ref)

    p = p_ref[...]
    psum = p[0] + p[1]
    den = den_ref[...]
    d4 = den[0] + den[1]
    r8 = jnp.repeat(1.0 / (d4 + 1e-30), 2, axis=1)
    scaled = psum * r8.T[:, :, None]
    emb = scaled.transpose(1, 0, 2).reshape(NB, HC) + b_ref[...]
    emb_ref[...] = emb

    b = batch_ref[...].reshape(1, NB)
    gids = lax.broadcasted_iota(jnp.int32, (G, NB), 0)
    onehot = jnp.where(gids == b, 1.0, 0.0)
    pool_ref[...] += jnp.dot(onehot, emb, preferred_element_type=jnp.float32)

    @pl.when(i == NBLK - 1)
    def _():
        h1 = jnp.maximum(
            jnp.dot(pool_ref[...], fcW0_ref[...],
                    preferred_element_type=jnp.float32) + fcb0_ref[...], 0.0)
        out_ref[...] = (jnp.dot(h1, fcW1_ref[...],
                                preferred_element_type=jnp.float32)
                        + fcb1_ref[...])


def _tc2(partial, denom, b1, batch, fcW0, fcb0, fcW1, fcb1):
    return pl.pallas_call(
        _tc2_body,
        grid=(NBLK,),
        in_specs=[
            pl.BlockSpec((2, KB, NB, 128), lambda i: (0, 0, i, 0)),
            pl.BlockSpec((2, NB, H), lambda i: (0, i, 0)),
            pl.BlockSpec((1, HC), lambda i: (0, 0)),
            pl.BlockSpec((1, 1, NB), lambda i: (i, 0, 0)),
            pl.BlockSpec((HC, FC1), lambda i: (0, 0)),
            pl.BlockSpec((1, FC1), lambda i: (0, 0)),
            pl.BlockSpec((FC1, FC2), lambda i: (0, 0)),
            pl.BlockSpec((1, FC2), lambda i: (0, 0)),
        ],
        out_specs=[
            pl.BlockSpec((NB, HC), lambda i: (i, 0)),
            pl.BlockSpec((G, FC2), lambda i: (0, 0)),
        ],
        out_shape=[
            jax.ShapeDtypeStruct((N, HC), jnp.float32),
            jax.ShapeDtypeStruct((G, FC2), jnp.float32),
        ],
        scratch_shapes=[pltpu.VMEM((G, HC), jnp.float32)],
    )(partial, denom, b1.reshape(1, HC),
      batch.astype(jnp.int32).reshape(NBLK, 1, NB),
      fcW0, fcb0.reshape(1, FC1), fcW1, fcb1.reshape(1, FC2))


NC = 2
NS = 16
NP = 10240
NPT = NP // NS
ZR = 32
DROWS = NP // 8


def _vgather(v, idx):
    return lax.gather(
        v, idx.reshape(16, 1),
        dimension_numbers=lax.GatherDimensionNumbers(
            offset_dims=(), collapsed_slice_dims=(0,), start_index_map=(0,)),
        slice_sizes=(1,), mode=lax.GatherScatterMode.PROMISE_IN_BOUNDS)


def _lane_bcast(v, i):
    return _vgather(v, jnp.full((16,), i, jnp.int32))


GRPA = 314
GRPB = 316


def _sc_edge_body(xl_hbm, atab_hbm, src_hbm, dst_hbm,
                  partial_hbm, denom_hbm, ex_hbm,
                  src_v, dst_v, gb0, gb1, gb2, gb3, msg0, msg1, exbuf, exbuf2,
                  zbuf, exstg, pbuf, acc_sh, den_sh,
                  sm0, sm1, sm2, sm3, sm4, sm5):
    c = lax.axis_index("c")
    s = lax.axis_index("s")
    w = s * NC + c
    iota16 = lax.iota(jnp.int32, 16)
    zero16 = jnp.zeros((16,), jnp.float32)
    gbufs = (gb0, gb1, gb2, gb3)
    gsems = (sm0, sm1, sm2, sm3)
    msgbufs = (msg0, msg1)
    msems = (sm4, sm5)

    pltpu.sync_copy(src_hbm.at[pl.ds(w * EWP, EWP)], src_v)
    pltpu.sync_copy(dst_hbm.at[pl.ds(w * EWP, EWP)], dst_v)

    def _zrow(i, carry):
        for k in range(8):
            zbuf[i, pl.ds(k * 16, 16)] = zero16
        return carry

    lax.fori_loop(0, ZR, _zrow, 0)
    for i in range(16):
        for k in range(8):
            exbuf[pl.ds(i * 128 + k * 16, 16)] = zero16
            exbuf2[i, pl.ds(k * 16, 16)] = zero16

    def _wait16(sem):
        pltpu.make_async_copy(xl_hbm.at[pl.ds(0, 16)], msg0, sem).wait()

    dzr = DROWS // NS
    for t in range(dzr // ZR + (1 if dzr % ZR else 0)):
        ln = min(ZR, dzr - t * ZR)
        pltpu.sync_copy(zbuf.at[pl.ds(0, ln)],
                        den_sh.at[pl.ds(s * dzr + t * ZR, ln)])
    plsc.subcore_barrier()

    def _issue_a(g, bs, bd, ss, sd):
        srcv = src_v[pl.ds(g * 16, 16)]
        dstv = dst_v[pl.ds(g * 16, 16)]
        pltpu.async_copy(atab_hbm.at[srcv // 8], bs, ss)
        pltpu.async_copy(atab_hbm.at[dstv // 8], bd, sd)

    _issue_a(0, gb0, gb1, sm0, sm1)
    _issue_a(1, gb2, gb3, sm2, sm3)

    def _grp_den(g2, carry):
        for b in range(2):
            g = g2 * 2 + b
            bs, bd = gbufs[2 * b], gbufs[2 * b + 1]
            ss, sd = gsems[2 * b], gsems[2 * b + 1]
            _wait16(ss)
            _wait16(sd)
            dstv = dst_v[pl.ds(g * 16, 16)]
            srcv = src_v[pl.ds(g * 16, 16)]
            gmask = jnp.where(g * 16 + iota16 < EVALID,
                              jnp.float32(1.0), jnp.float32(0.0))
            base = (g % FBG) * 16
            ls = (srcv % 8) * 16
            ld = (dstv % 8) * 16
            for h in range(H):
                a_s = plsc.load_gather(bs, [iota16, ls + h])
                a_d = plsc.load_gather(bd, [iota16, ld + (H + h)])
                al = a_s + a_d
                al = jnp.where(al > 0, al, 0.2 * al)
                exh = jnp.exp(al) * gmask
                exstg[pl.ds(h * (FBG * 16) + base, 16)] = exh
                plsc.store_scatter(exbuf, [iota16 * 128 + ld + h], exh)

            @pl.when(g + 2 < GRPA)
            def _():
                _issue_a(g + 2, bs, bd, ss, sd)

            for i in range(16):
                for k in range(8):
                    exbuf2[i, pl.ds(k * 16, 16)] = \
                        exbuf[pl.ds(i * 128 + k * 16, 16)]
            pltpu.sync_copy(exbuf2, den_sh.at[dstv // 8], add=True)
            for h in range(H):
                plsc.store_scatter(exbuf, [iota16 * 128 + ld + h], zero16)

            @pl.when(g % FBG == FBG - 1)
            def _():
                blk = g // FBG
                for h in range(H):
                    pltpu.sync_copy(
                        exstg.at[pl.ds(h * (FBG * 16), FBG * 16)],
                        ex_hbm.at[pl.ds((w * H + h) * EWP + blk * (FBG * 16),
                                        FBG * 16)])
        return carry

    lax.fori_loop(0, GRPA // 2, _grp_den, 0)
    for h in range(H):
        pltpu.sync_copy(
            exstg.at[pl.ds(h * (FBG * 16), FBG * 16)],
            ex_hbm.at[pl.ds((w * H + h) * EWP + (GRPA // FBG) * (FBG * 16),
                            FBG * 16)])
    plsc.subcore_barrier()
    for t in range(dzr // ZR + (1 if dzr % ZR else 0)):
        ln = min(ZR, dzr - t * ZR)
        pltpu.sync_copy(den_sh.at[pl.ds(s * dzr + t * ZR, ln)],
                        denom_hbm.at[c, pl.ds(s * dzr + t * ZR, ln)])

    def _pass(j, carry):
        h = j // 2
        pltpu.sync_copy(ex_hbm.at[pl.ds((w * H + h) * EWP, EWP)], pbuf)
        for t in range(NPT // ZR):
            pltpu.sync_copy(zbuf, acc_sh.at[pl.ds(s * NPT + t * ZR, ZR)])
        plsc.subcore_barrier()

        def _issue_b(g, buf, sem):
            srcv = src_v[pl.ds(g * 16, 16)]
            pltpu.async_copy(xl_hbm.at[srcv + j * N], buf, sem)

        for b in range(4):
            _issue_b(b, gbufs[b], gsems[b])

        def _grp(g4, carry2):
            for b in range(4):
                g = g4 * 4 + b
                buf, sem = gbufs[b], gsems[b]
                m = b % 2
                mbuf, msem = msgbufs[m], msems[m]
                dstv = dst_v[pl.ds(g * 16, 16)]
                w16 = pbuf[pl.ds(g * 16, 16)]
                w16 = jnp.where(g * 16 + iota16 < EVALID, w16, 0.0)
                wis = [_lane_bcast(w16, i) for i in range(16)]
                _wait16(sem)

                @pl.when(g >= 2)
                def _():
                    _wait16(msem)

                for i in range(16):
                    for k in range(8):
                        mbuf[i, pl.ds(k * 16, 16)] = \
                            buf[i, pl.ds(k * 16, 16)] * wis[i]

                _issue_b(g + 4, buf, sem)
                pltpu.async_copy(mbuf, acc_sh.at[dstv], msem, add=True)
            return carry2

        lax.fori_loop(0, GRPB // 4, _grp, 0)
        for b in range(4):
            _wait16(gsems[b])
        _wait16(msems[0])
        _wait16(msems[1])
        plsc.subcore_barrier()
        pltpu.sync_copy(acc_sh.at[pl.ds(s * NPT, NPT)],
                        partial_hbm.at[c, j, pl.ds(s * NPT, NPT)])
        plsc.subcore_barrier()
        return carry

    lax.fori_loop(0, KB, _pass, 0)


@functools.partial(
    pl.kernel,
    mesh=plsc.VectorSubcoreMesh(core_axis_name="c", subcore_axis_name="s"),
    compiler_params=pltpu.CompilerParams(needs_layout_passes=False),
    out_type=[
        jax.ShapeDtypeStruct((NC, KB, NP, 128), jnp.float32),
        jax.ShapeDtypeStruct((NC, DROWS, 128), jnp.float32),
        jax.ShapeDtypeStruct((NW * H * EWP,), jnp.float32),
    ],
    scratch_types=[
        pltpu.VMEM((EWP,), jnp.int32),
        pltpu.VMEM((EWP,), jnp.int32),
        pltpu.VMEM((16, 128), jnp.float32),
        pltpu.VMEM((16, 128), jnp.float32),
        pltpu.VMEM((16, 128), jnp.float32),
        pltpu.VMEM((16, 128), jnp.float32),
        pltpu.VMEM((16, 128), jnp.float32),
        pltpu.VMEM((16, 128), jnp.float32),
        pltpu.VMEM((16 * 128,), jnp.float32),
        pltpu.VMEM((16, 128), jnp.float32),
        pltpu.VMEM((ZR, 128), jnp.float32),
        pltpu.VMEM((H * FBG * 16,), jnp.float32),
        pltpu.VMEM((EWP,), jnp.float32),
        pltpu.VMEM_SHARED((NP, 128), jnp.float32),
        pltpu.VMEM_SHARED((DROWS, 128), jnp.float32),
        pltpu.SemaphoreType.DMA,
        pltpu.SemaphoreType.DMA,
        pltpu.SemaphoreType.DMA,
        pltpu.SemaphoreType.DMA,
        pltpu.SemaphoreType.DMA,
        pltpu.SemaphoreType.DMA,
    ],
)
def _sc_edge(xl_hbm, atab_hbm, src_hbm, dst_hbm,
             partial_hbm, denom_hbm, ex_hbm, *scratch):
    _sc_edge_body(xl_hbm, atab_hbm, src_hbm, dst_hbm,
                  partial_hbm, denom_hbm, ex_hbm, *scratch)


def _edge_phase(xl, atab, src3, dst3):
    partial, den_packed, _ex = _sc_edge(xl, atab.reshape(N // 8, 128),
                                        src3, dst3)
    den = den_packed.reshape(NC, NP, 16)[:, :, :H]
    return partial, den


def kernel(x_n, edge_index, batch, W0, att_src0, att_dst0, b0,
           W1, att_src1, att_dst1, b1, fcW0, fcb0, fcW1, fcb1):
    src = edge_index[0].astype(jnp.int32)
    dst = edge_index[1].astype(jnp.int32)
    src3 = jnp.pad(src.reshape(NW, EVALID),
                   ((0, 0), (0, EWP - EVALID))).reshape(NW * EWP)
    dst3 = jnp.pad(dst.reshape(NW, EVALID),
                   ((0, 0), (0, EWP - EVALID))).reshape(NW * EWP)

    xl0, atab0 = _tc0(x_n, W0, att_src0.reshape(HC), att_dst0.reshape(HC))
    p0, den0 = _edge_phase(xl0.reshape(KB * N, 128), atab0, src3, dst3)
    xl1, atab1 = _tc1(p0, den0, b0, W1,
                      att_src1.reshape(HC), att_dst1.reshape(HC))
    p1, den1 = _edge_phase(xl1.reshape(KB * N, 128), atab1, src3, dst3)
    node_embeddings, uB = _tc2(p1, den1, b1, batch, fcW0, fcb0, fcW1, fcb1)
    return (uB, node_embeddings)


# speedup vs baseline: 1.4276x; 1.4276x over previous
import functools

import jax
import jax.numpy as jnp
from jax import lax
from jax.experimental import pallas as pl
from jax.experimental.pallas import tpu as pltpu
from jax.experimental.pallas import tpu_sc as plsc

N = 10000
E = 160000
D = 256
H = 4
C = 256
HC = H * C
G = 16
FC1 = 512
FC2 = 128

NB = 1000
NBLK = N // NB
KB = 8
NW = 32
EVALID = E // NW
EWP = 5120
GRP = 313
AW = 16
FBG = 32


def _tc0_body(x_ref, W_ref, fs_ref, fd_ref, xl_ref, atab_ref):
    xb = x_ref[...]
    xl = jnp.dot(xb, W_ref[...], preferred_element_type=jnp.float32)
    xl_ref[...] = xl.reshape(NB, KB, 128).transpose(1, 0, 2)
    rows = lax.broadcasted_iota(jnp.int32, (HC, AW), 0) // C
    cols = lax.broadcasted_iota(jnp.int32, (HC, AW), 1)
    fs = fs_ref[...].reshape(HC, 1)
    fd = fd_ref[...].reshape(HC, 1)
    WS = jnp.where(cols == rows, fs, 0.0) + jnp.where(cols == rows + H, fd, 0.0)
    atab_ref[...] = jnp.dot(xl, WS, preferred_element_type=jnp.float32)


def _tc0(x, W0, fs0, fd0):
    return pl.pallas_call(
        _tc0_body,
        grid=(NBLK,),
        in_specs=[
            pl.BlockSpec((NB, D), lambda i: (i, 0)),
            pl.BlockSpec((D, HC), lambda i: (0, 0)),
            pl.BlockSpec((1, HC), lambda i: (0, 0)),
            pl.BlockSpec((1, HC), lambda i: (0, 0)),
        ],
        out_specs=[
            pl.BlockSpec((KB, NB, 128), lambda i: (0, i, 0)),
            pl.BlockSpec((NB, AW), lambda i: (i, 0)),
        ],
        out_shape=[
            jax.ShapeDtypeStruct((KB, N, 128), jnp.float32),
            jax.ShapeDtypeStruct((N, AW), jnp.float32),
        ],
    )(x, W0, fs0.reshape(1, HC), fd0.reshape(1, HC))


def _tc1_body(p_ref, den_ref, b_ref, W_ref, fs_ref, fd_ref,
              xl_ref, atab_ref, acc_ref):
    j = pl.program_id(1)

    @pl.when(j == 0)
    def _():
        acc_ref[...] = jnp.zeros_like(acc_ref)

    p = p_ref[...]
    psum = p[0, 0] + p[1, 0]
    den = den_ref[...]
    d4 = den[0] + den[1]
    h_idx = j // 2
    cols = lax.broadcasted_iota(jnp.int32, (NB, H), 1)
    dh = jnp.sum(jnp.where(cols == h_idx, d4, 0.0), axis=1, keepdims=True)
    normed = psum * (1.0 / (dh + 1e-30)) + b_ref[...]
    hblk = jnp.where(normed > 0, normed, jnp.exp(normed) - 1.0)
    acc_ref[...] += jnp.dot(hblk, W_ref[...], preferred_element_type=jnp.float32)

    @pl.when(j == KB - 1)
    def _():
        xl1 = acc_ref[...]
        xl_ref[...] = xl1.reshape(NB, KB, 128).transpose(1, 0, 2)
        rows = lax.broadcasted_iota(jnp.int32, (HC, AW), 0) // C
        wcols = lax.broadcasted_iota(jnp.int32, (HC, AW), 1)
        fs = fs_ref[...].reshape(HC, 1)
        fd = fd_ref[...].reshape(HC, 1)
        WS = jnp.where(wcols == rows, fs, 0.0) \
           + jnp.where(wcols == rows + H, fd, 0.0)
        atab_ref[...] = jnp.dot(xl1, WS, preferred_element_type=jnp.float32)


def _tc1(partial, denom, b0, W1, fs1, fd1):
    return pl.pallas_call(
        _tc1_body,
        grid=(NBLK, KB),
        in_specs=[
            pl.BlockSpec((2, 1, NB, 128), lambda i, j: (0, j, i, 0)),
            pl.BlockSpec((2, NB, H), lambda i, j: (0, i, 0)),
            pl.BlockSpec((1, 128), lambda i, j: (0, j)),
            pl.BlockSpec((128, HC), lambda i, j: (j, 0)),
            pl.BlockSpec((1, HC), lambda i, j: (0, 0)),
            pl.BlockSpec((1, HC), lambda i, j: (0, 0)),
        ],
        out_specs=[
            pl.BlockSpec((KB, NB, 128), lambda i, j: (0, i, 0)),
            pl.BlockSpec((NB, AW), lambda i, j: (i, 0)),
        ],
        out_shape=[
            jax.ShapeDtypeStruct((KB, N, 128), jnp.float32),
            jax.ShapeDtypeStruct((N, AW), jnp.float32),
        ],
        scratch_shapes=[
            pltpu.VMEM((NB, HC), jnp.float32),
        ],
    )(partial, denom, b0.reshape(1, HC), W1,
      fs1.reshape(1, HC), fd1.reshape(1, HC))


def _tc2_body(p_ref, den_ref, b_ref, batch_ref, fcW0_ref, fcb0_ref,
              fcW1_ref, fcb1_ref, emb_ref, out_ref, pool_ref):
    i = pl.program_id(0)

    @pl.when(i == 0)
    def _():
        pool_ref[...] = jnp.zeros_like(pool_ref)

    p = p_ref[...]
    psum = p[0] + p[1]
    den = den_ref[...]
    d4 = den[0] + den[1]
    r8 = jnp.repeat(1.0 / (d4 + 1e-30), 2, axis=1)
    scaled = psum * r8.T[:, :, None]
    emb = scaled.transpose(1, 0, 2).reshape(NB, HC) + b_ref[...]
    emb_ref[...] = emb

    b = batch_ref[...].reshape(1, NB)
    gids = lax.broadcasted_iota(jnp.int32, (G, NB), 0)
    onehot = jnp.where(gids == b, 1.0, 0.0)
    pool_ref[...] += jnp.dot(onehot, emb, preferred_element_type=jnp.float32)

    @pl.when(i == NBLK - 1)
    def _():
        h1 = jnp.maximum(
            jnp.dot(pool_ref[...], fcW0_ref[...],
                    preferred_element_type=jnp.float32) + fcb0_ref[...], 0.0)
        out_ref[...] = (jnp.dot(h1, fcW1_ref[...],
                                preferred_element_type=jnp.float32)
                        + fcb1_ref[...])


def _tc2(partial, denom, b1, batch, fcW0, fcb0, fcW1, fcb1):
    return pl.pallas_call(
        _tc2_body,
        grid=(NBLK,),
        in_specs=[
            pl.BlockSpec((2, KB, NB, 128), lambda i: (0, 0, i, 0)),
            pl.BlockSpec((2, NB, H), lambda i: (0, i, 0)),
            pl.BlockSpec((1, HC), lambda i: (0, 0)),
            pl.BlockSpec((1, 1, NB), lambda i: (i, 0, 0)),
            pl.BlockSpec((HC, FC1), lambda i: (0, 0)),
            pl.BlockSpec((1, FC1), lambda i: (0, 0)),
            pl.BlockSpec((FC1, FC2), lambda i: (0, 0)),
            pl.BlockSpec((1, FC2), lambda i: (0, 0)),
        ],
        out_specs=[
            pl.BlockSpec((NB, HC), lambda i: (i, 0)),
            pl.BlockSpec((G, FC2), lambda i: (0, 0)),
        ],
        out_shape=[
            jax.ShapeDtypeStruct((N, HC), jnp.float32),
            jax.ShapeDtypeStruct((G, FC2), jnp.float32),
        ],
        scratch_shapes=[pltpu.VMEM((G, HC), jnp.float32)],
    )(partial, denom, b1.reshape(1, HC),
      batch.astype(jnp.int32).reshape(NBLK, 1, NB),
      fcW0, fcb0.reshape(1, FC1), fcW1, fcb1.reshape(1, FC2))


NC = 2
NS = 16
NP = 10240
NPT = NP // NS
ZR = 32
DROWS = NP // 8


def _vgather(v, idx):
    return lax.gather(
        v, idx.reshape(16, 1),
        dimension_numbers=lax.GatherDimensionNumbers(
            offset_dims=(), collapsed_slice_dims=(0,), start_index_map=(0,)),
        slice_sizes=(1,), mode=lax.GatherScatterMode.PROMISE_IN_BOUNDS)


def _lane_bcast(v, i):
    return _vgather(v, jnp.full((16,), i, jnp.int32))


GRPA = 314
GRPB = 316


def _sc_edge_body(xl_hbm, atab_hbm, src_hbm, dst_hbm,
                  partial_hbm, denom_hbm, ex_hbm,
                  src_v, dst_v, gb0, gb1, gb2, gb3, msg0, msg1, exbuf, exbuf2,
                  zbuf, exstg, pbuf, acc_sh, den_sh,
                  sm0, sm1, sm2, sm3, sm4, sm5):
    c = lax.axis_index("c")
    s = lax.axis_index("s")
    w = s * NC + c
    iota16 = lax.iota(jnp.int32, 16)
    zero16 = jnp.zeros((16,), jnp.float32)
    gbufs = (gb0, gb1, gb2, gb3)
    gsems = (sm0, sm1, sm2, sm3)
    msgbufs = (msg0, msg1)
    msems = (sm4, sm5)

    pltpu.sync_copy(src_hbm.at[pl.ds(w * EWP, EWP)], src_v)
    pltpu.sync_copy(dst_hbm.at[pl.ds(w * EWP, EWP)], dst_v)

    def _zrow(i, carry):
        for k in range(8):
            zbuf[i, pl.ds(k * 16, 16)] = zero16
        return carry

    lax.fori_loop(0, ZR, _zrow, 0)
    for i in range(16):
        for k in range(8):
            exbuf[pl.ds(i * 128 + k * 16, 16)] = zero16
            exbuf2[i, pl.ds(k * 16, 16)] = zero16

    def _wait16(sem):
        pltpu.make_async_copy(xl_hbm.at[pl.ds(0, 16)], msg0, sem).wait()

    dzr = DROWS // NS
    for t in range(dzr // ZR + (1 if dzr % ZR else 0)):
        ln = min(ZR, dzr - t * ZR)
        pltpu.sync_copy(zbuf.at[pl.ds(0, ln)],
                        den_sh.at[pl.ds(s * dzr + t * ZR, ln)])
    plsc.subcore_barrier()

    def _issue_a(g, bs, bd, ss, sd):
        srcv = src_v[pl.ds(g * 16, 16)]
        dstv = dst_v[pl.ds(g * 16, 16)]
        pltpu.async_copy(atab_hbm.at[srcv // 8], bs, ss)
        pltpu.async_copy(atab_hbm.at[dstv // 8], bd, sd)

    _issue_a(0, gb0, gb1, sm0, sm1)
    _issue_a(1, gb2, gb3, sm2, sm3)

    def _grp_den(g2, carry):
        for b in range(2):
            g = g2 * 2 + b
            bs, bd = gbufs[2 * b], gbufs[2 * b + 1]
            ss, sd = gsems[2 * b], gsems[2 * b + 1]
            _wait16(ss)
            _wait16(sd)
            dstv = dst_v[pl.ds(g * 16, 16)]
            srcv = src_v[pl.ds(g * 16, 16)]
            gmask = jnp.where(g * 16 + iota16 < EVALID,
                              jnp.float32(1.0), jnp.float32(0.0))
            base = (g % FBG) * 16
            ls = (srcv % 8) * 16
            ld = (dstv % 8) * 16
            for h in range(H):
                a_s = plsc.load_gather(bs, [iota16, ls + h])
                a_d = plsc.load_gather(bd, [iota16, ld + (H + h)])
                al = a_s + a_d
                al = jnp.where(al > 0, al, 0.2 * al)
                exh = jnp.exp(al) * gmask
                exstg[pl.ds(h * (FBG * 16) + base, 16)] = exh
                plsc.store_scatter(exbuf, [iota16 * 128 + ld + h], exh)

            @pl.when(g + 2 < GRPA)
            def _():
                _issue_a(g + 2, bs, bd, ss, sd)

            for i in range(16):
                for k in range(8):
                    exbuf2[i, pl.ds(k * 16, 16)] = \
                        exbuf[pl.ds(i * 128 + k * 16, 16)]
            pltpu.sync_copy(exbuf2, den_sh.at[dstv // 8], add=True)
            for h in range(H):
                plsc.store_scatter(exbuf, [iota16 * 128 + ld + h], zero16)

            @pl.when(g % FBG == FBG - 1)
            def _():
                blk = g // FBG
                for h in range(H):
                    pltpu.sync_copy(
                        exstg.at[pl.ds(h * (FBG * 16), FBG * 16)],
                        ex_hbm.at[pl.ds((w * H + h) * EWP + blk * (FBG * 16),
                                        FBG * 16)])
        return carry

    lax.fori_loop(0, GRPA // 2, _grp_den, 0)
    for h in range(H):
        pltpu.sync_copy(
            exstg.at[pl.ds(h * (FBG * 16), FBG * 16)],
            ex_hbm.at[pl.ds((w * H + h) * EWP + (GRPA // FBG) * (FBG * 16),
                            FBG * 16)])
    plsc.subcore_barrier()
    for t in range(dzr // ZR + (1 if dzr % ZR else 0)):
        ln = min(ZR, dzr - t * ZR)
        pltpu.sync_copy(den_sh.at[pl.ds(s * dzr + t * ZR, ln)],
                        denom_hbm.at[c, pl.ds(s * dzr + t * ZR, ln)])

    def _pass(j, carry):
        h = j // 2
        pltpu.sync_copy(ex_hbm.at[pl.ds((w * H + h) * EWP, EWP)], pbuf)
        for t in range(NPT // ZR):
            pltpu.sync_copy(zbuf, acc_sh.at[pl.ds(s * NPT + t * ZR, ZR)])
        plsc.subcore_barrier()

        def _issue_b(g, buf, sem):
            srcv = src_v[pl.ds(g * 16, 16)]
            pltpu.async_copy(xl_hbm.at[srcv + j * N], buf, sem)

        for b in range(4):
            _issue_b(b, gbufs[b], gsems[b])

        def _grp(g4, carry2):
            for b in range(4):
                g = g4 * 4 + b
                buf, sem = gbufs[b], gsems[b]
                m = b % 2
                mbuf, msem = msgbufs[m], msems[m]
                _wait16(sem)

                @pl.when(g >= 2)
                def _():
                    _wait16(msem)

                dstv = dst_v[pl.ds(g * 16, 16)]
                w16 = pbuf[pl.ds(g * 16, 16)]
                w16 = jnp.where(g * 16 + iota16 < EVALID, w16, 0.0)
                for i in range(16):
                    wi = _lane_bcast(w16, i)
                    for k in range(8):
                        mbuf[i, pl.ds(k * 16, 16)] = \
                            buf[i, pl.ds(k * 16, 16)] * wi

                @pl.when(g + 4 < GRPB)
                def _():
                    _issue_b(g + 4, buf, sem)

                pltpu.async_copy(mbuf, acc_sh.at[dstv], msem, add=True)
            return carry2

        lax.fori_loop(0, GRPB // 4, _grp, 0)
        _wait16(msems[0])
        _wait16(msems[1])
        plsc.subcore_barrier()
        pltpu.sync_copy(acc_sh.at[pl.ds(s * NPT, NPT)],
                        partial_hbm.at[c, j, pl.ds(s * NPT, NPT)])
        plsc.subcore_barrier()
        return carry

    lax.fori_loop(0, KB, _pass, 0)


@functools.partial(
    pl.kernel,
    mesh=plsc.VectorSubcoreMesh(core_axis_name="c", subcore_axis_name="s"),
    compiler_params=pltpu.CompilerParams(needs_layout_passes=False),
    out_type=[
        jax.ShapeDtypeStruct((NC, KB, NP, 128), jnp.float32),
        jax.ShapeDtypeStruct((NC, DROWS, 128), jnp.float32),
        jax.ShapeDtypeStruct((NW * H * EWP,), jnp.float32),
    ],
    scratch_types=[
        pltpu.VMEM((EWP,), jnp.int32),
        pltpu.VMEM((EWP,), jnp.int32),
        pltpu.VMEM((16, 128), jnp.float32),
        pltpu.VMEM((16, 128), jnp.float32),
        pltpu.VMEM((16, 128), jnp.float32),
        pltpu.VMEM((16, 128), jnp.float32),
        pltpu.VMEM((16, 128), jnp.float32),
        pltpu.VMEM((16, 128), jnp.float32),
        pltpu.VMEM((16 * 128,), jnp.float32),
        pltpu.VMEM((16, 128), jnp.float32),
        pltpu.VMEM((ZR, 128), jnp.float32),
        pltpu.VMEM((H * FBG * 16,), jnp.float32),
        pltpu.VMEM((EWP,), jnp.float32),
        pltpu.VMEM_SHARED((NP, 128), jnp.float32),
        pltpu.VMEM_SHARED((DROWS, 128), jnp.float32),
        pltpu.SemaphoreType.DMA,
        pltpu.SemaphoreType.DMA,
        pltpu.SemaphoreType.DMA,
        pltpu.SemaphoreType.DMA,
        pltpu.SemaphoreType.DMA,
        pltpu.SemaphoreType.DMA,
    ],
)
def _sc_edge(xl_hbm, atab_hbm, src_hbm, dst_hbm,
             partial_hbm, denom_hbm, ex_hbm, *scratch):
    _sc_edge_body(xl_hbm, atab_hbm, src_hbm, dst_hbm,
                  partial_hbm, denom_hbm, ex_hbm, *scratch)


def _edge_phase(xl, atab, src3, dst3):
    partial, den_packed, _ex = _sc_edge(xl, atab.reshape(N // 8, 128),
                                        src3, dst3)
    den = den_packed.reshape(NC, NP, 16)[:, :, :H]
    return partial, den


def kernel(x_n, edge_index, batch, W0, att_src0, att_dst0, b0,
           W1, att_src1, att_dst1, b1, fcW0, fcb0, fcW1, fcb1):
    src = edge_index[0].astype(jnp.int32)
    dst = edge_index[1].astype(jnp.int32)
    src3 = jnp.pad(src.reshape(NW, EVALID),
                   ((0, 0), (0, EWP - EVALID))).reshape(NW * EWP)
    dst3 = jnp.pad(dst.reshape(NW, EVALID),
                   ((0, 0), (0, EWP - EVALID))).reshape(NW * EWP)

    xl0, atab0 = _tc0(x_n, W0, att_src0.reshape(HC), att_dst0.reshape(HC))
    p0, den0 = _edge_phase(xl0.reshape(KB * N, 128), atab0, src3, dst3)
    xl1, atab1 = _tc1(p0, den0, b0, W1,
                      att_src1.reshape(HC), att_dst1.reshape(HC))
    p1, den1 = _edge_phase(xl1.reshape(KB * N, 128), atab1, src3, dst3)
    node_embeddings, uB = _tc2(p1, den1, b1, batch, fcW0, fcb0, fcW1, fcb1)
    return (uB, node_embeddings)
